# Initial kernel scaffold; baseline (speedup 1.0000x reference)
import jax
import jax.numpy as jnp
from jax.experimental import pallas as pl


def kernel(x, edge_score, node_score, fc1_W, fc1_b, fc2_W, fc2_b, gnn_W0, gnn_b0, gnn_W1, gnn_b1, gnn_W2, gnn_b2):
    raise NotImplementedError("write your pallas kernel here")


# speedup vs baseline: 13.3673x; 13.3673x over previous
import functools

import jax
import jax.numpy as jnp
from jax import lax
from jax.experimental import pallas as pl
from jax.experimental.pallas import tpu as pltpu
from jax.experimental.pallas import tpu_sc as plsc

B, N, S, H = 8, 1024, 128, 64
L, K0, DSN, DSE = 3, 8, 2, 2

_HIGH = jax.lax.Precision.HIGHEST
_NEG_INF = -float("inf")


def _topk_cols(s, kk, off):
  rows, width = s.shape
  li = lax.broadcasted_iota(jnp.int32, (rows, width), 1)
  big = jnp.int32(1 << 30)
  cols = []
  for _ in range(kk):
    m = jnp.max(s, axis=1, keepdims=True)
    cand = jnp.where(s == m, li, big)
    a = jnp.min(cand, axis=1, keepdims=True)
    cols.append(a + off)
    s = jnp.where(li == a, _NEG_INF, s)
  return jnp.concatenate(cols, axis=1)


def _bitonic_desc(key, payloads, nseg):
  n = nseg * 128
  shape = key.shape
  pos = (lax.broadcasted_iota(jnp.int32, shape, 1) * 128
         + lax.broadcasted_iota(jnp.int32, shape, 2))
  lane = lax.broadcasted_iota(jnp.int32, shape, 2)
  k = 2
  while k <= n:
    j = k // 2
    while j >= 1:
      if j >= 128:
        sj = j // 128
        perm = [seg ^ sj for seg in range(nseg)]
        pk = jnp.concatenate([key[:, q:q + 1, :] for q in perm], axis=1)
        pp = [jnp.concatenate([p[:, q:q + 1, :] for q in perm], axis=1)
              for p in payloads]
      else:
        gidx = lane ^ j
        pk = jnp.take_along_axis(key, gidx, axis=2)
        pp = [jnp.take_along_axis(p, gidx, axis=2) for p in payloads]
      w = (key > pk) | ((key == pk) & (payloads[0] < pp[0]))
      tw = ((pos & j) == 0) == ((pos & k) == 0)
      keep = w == tw
      key = jnp.where(keep, key, pk)
      payloads = [jnp.where(keep, p, q) for p, q in zip(payloads, pp)]
      j //= 2
    k *= 2
  return key, payloads


def _bitonic_asc_vals(x):
  rows, n = x.shape
  lane = lax.broadcasted_iota(jnp.int32, (rows, n), 1)
  k = 2
  while k <= n:
    j = k // 2
    while j >= 1:
      px = jnp.take_along_axis(x, lane ^ j, axis=1)
      low = (lane & j) == 0
      w = (x < px) | ((x == px) & low)
      tw = low == ((lane & k) == 0)
      x = jnp.where(w == tw, x, px)
      j //= 2
    k *= 2
  return x


def _node_select_body(ln_ref, gn0_ref, gn1_ref, i0_ref, i1_ref, sel_ref):
  ln = ln_ref[...]
  key0 = ln + gn0_ref[...]
  idx = (lax.broadcasted_iota(jnp.int32, key0.shape, 1) * 128
         + lax.broadcasted_iota(jnp.int32, key0.shape, 2))
  _, (idx_s, ln_s) = _bitonic_desc(key0, [idx, ln], 8)
  i0 = idx_s[:, :4, :]
  ln1 = ln_s[:, :4, :]
  key1 = ln1 + gn1_ref[...]
  idx2 = (lax.broadcasted_iota(jnp.int32, key1.shape, 1) * 128
          + lax.broadcasted_iota(jnp.int32, key1.shape, 2))
  _, (idx2_s, sel_s) = _bitonic_desc(key1, [idx2, i0], 4)
  i0_ref[...] = i0
  i1_ref[...] = idx2_s[:, :2, :]
  sel_ref[...] = sel_s[:, :2, :]


def _node_select(ln3, gn03, gn13):
  out = pl.pallas_call(
      _node_select_body,
      out_shape=(
          jax.ShapeDtypeStruct((B, 4, 128), jnp.int32),
          jax.ShapeDtypeStruct((B, 2, 128), jnp.int32),
          jax.ShapeDtypeStruct((B, 2, 128), jnp.int32),
      ),
  )(ln3, gn03, gn13)
  i0 = out[0].reshape(B, N // 2)
  i1 = out[1].reshape(B, N // 4)
  sel = out[2].reshape(B, N // 4)
  return i0, i1, sel


def _edge_topk_body(kk, rows_per_batch, add_batch_off, ls_ref, g_ref, o_ref):
  s = ls_ref[...] + g_ref[...]
  if add_batch_off:
    pid = pl.program_id(0)
    blk_rows = s.shape[0]
    off = (pid * blk_rows // rows_per_batch) * rows_per_batch
  else:
    off = 0
  o_ref[...] = _topk_cols(s, kk, off)


def _edge_topk(ls2, g2, kk, rows_per_batch, add_batch_off, blk_rows=128):
  total, width = ls2.shape
  body = functools.partial(_edge_topk_body, kk, rows_per_batch, add_batch_off)
  return pl.pallas_call(
      body,
      grid=(total // blk_rows,),
      in_specs=[
          pl.BlockSpec((blk_rows, width), lambda i: (i, 0)),
          pl.BlockSpec((blk_rows, width), lambda i: (i, 0)),
      ],
      out_specs=pl.BlockSpec((blk_rows, kk), lambda i: (i, 0)),
      out_shape=jax.ShapeDtypeStruct((total, kk), jnp.int32),
  )(ls2, g2)


def _sort_fc1_body(x_ref, w1_ref, b1_ref, w0_ref, o_ref):
  xs = _bitonic_asc_vals(x_ref[...])
  h = lax.dot_general(xs, w1_ref[...], (((1,), (1,)), ((), ())),
                      precision=_HIGH) + b1_ref[...]
  o_ref[...] = lax.dot_general(h, w0_ref[...], (((1,), (1,)), ((), ())),
                               precision=_HIGH)


def _sort_fc1(x2, fc1_w, fc1_b, w0):
  blk = 512
  return pl.pallas_call(
      _sort_fc1_body,
      grid=(x2.shape[0] // blk,),
      in_specs=[
          pl.BlockSpec((blk, S), lambda i: (i, 0)),
          pl.BlockSpec((H, S), lambda i: (0, 0)),
          pl.BlockSpec((1, H), lambda i: (0, 0)),
          pl.BlockSpec((H, H), lambda i: (0, 0)),
      ],
      out_specs=pl.BlockSpec((blk, H), lambda i: (i, 0)),
      out_shape=jax.ShapeDtypeStruct((x2.shape[0], H), jnp.float32),
  )(x2, fc1_w, fc1_b, w0)


def _scale_mm_body(inv_deg, s_ref, b_ref, w_ref, o_ref):
  v = s_ref[...] * inv_deg + b_ref[...]
  xl = jnp.where(v >= 0, v, 0.01 * v)
  o_ref[...] = lax.dot_general(xl, w_ref[...], (((1,), (1,)), ((), ())),
                               precision=_HIGH)


def _scale_mm(sums, bias, w, inv_deg):
  blk = 512
  body = functools.partial(_scale_mm_body, inv_deg)
  return pl.pallas_call(
      body,
      grid=(sums.shape[0] // blk,),
      in_specs=[
          pl.BlockSpec((blk, H), lambda i: (i, 0)),
          pl.BlockSpec((1, H), lambda i: (0, 0)),
          pl.BlockSpec((H, H), lambda i: (0, 0)),
      ],
      out_specs=pl.BlockSpec((blk, H), lambda i: (i, 0)),
      out_shape=jax.ShapeDtypeStruct((sums.shape[0], H), jnp.float32),
  )(sums, bias, w)


def _final_body(s_ref, b2_ref, fw_ref, fb_ref, o_ref):
  v = s_ref[0] * (1.0 / 3.0) + b2_ref[...]
  x3 = jnp.where(v >= 0, v, 0.01 * v)
  fw = fw_ref[...]
  y_col = lax.dot_general(x3, fw, (((1,), (1,)), ((), ())),
                          precision=_HIGH)
  nf = y_col.shape[0]
  ones = jnp.full((nf, 1), 1.0, jnp.float32)
  cdims = (((1,), (1,)), ((), ()))
  d = (lax.dot_general(y_col, ones, cdims, precision=_HIGH)
       - lax.dot_general(ones, y_col, cdims, precision=_HIGH))
  o_ref[0] = 1.0 / (1.0 + jnp.exp(-d))


def _final(sum2_3d, b2, fc2_w, fc2_b):
  nf = sum2_3d.shape[1]
  return pl.pallas_call(
      _final_body,
      grid=(B,),
      in_specs=[
          pl.BlockSpec((1, nf, H), lambda i: (i, 0, 0)),
          pl.BlockSpec((1, H), lambda i: (0, 0)),
          pl.BlockSpec((1, H), lambda i: (0, 0)),
          pl.BlockSpec((1, 1), lambda i: (0, 0)),
      ],
      out_specs=pl.BlockSpec((1, nf, nf), lambda i: (i, 0, 0)),
      out_shape=jax.ShapeDtypeStruct((B, nf, nf), jnp.float32),
  )(sum2_3d, b2, fc2_w, fc2_b)


_NC = 2
_NS = 16
_NW = _NC * _NS


def _vadd(acc_ref, buf_ref, rows, cols):
  nv = cols // 16

  def body(r, c):
    for cc in range(nv):
      sl = pl.ds(cc * 16, 16)
      acc_ref[r, sl] = acc_ref[r, sl] + buf_ref[r, sl]
    return c

  lax.fori_loop(0, rows, body, 0)


def _vaddupdate_offset(idx_ref, n, off):
  def body(c, carry):
    sl = pl.ds(c * 16, 16)
    idx_ref[sl] = idx_ref[sl] + off
    return carry

  lax.fori_loop(0, n // 16, body, 0)


def _sc_gather_scores(ls_hbm, i0_hbm, sel_hbm, n1, n2):
  mesh = plsc.VectorSubcoreMesh(core_axis_name="c", subcore_axis_name="s", num_cores=_NC)
  rows1_per = B * n1 // _NW
  rows2_per = B * n2 // _NW
  ch = 64

  @functools.partial(
      pl.kernel,
      out_type=(
          jax.ShapeDtypeStruct((B * n1, n1), jnp.float32),
          jax.ShapeDtypeStruct((B * n2, n2), jnp.float32),
      ),
      mesh=mesh,
      compiler_params=pltpu.CompilerParams(needs_layout_passes=False, use_tc_tiling_on_sc=False),
      scratch_types=[
          pltpu.VMEM((ch, N), jnp.float32),
          pltpu.VMEM((N,), jnp.float32),
          pltpu.VMEM((ch, n1), jnp.float32),
          pltpu.VMEM((ch, n2), jnp.float32),
          pltpu.VMEM((ch,), jnp.int32),
          pltpu.VMEM((n1,), jnp.int32),
          pltpu.SemaphoreType.DMA,
      ],
  )
  def kern(ls_ref, i0_ref, sel_ref, o1_ref, o2_ref,
           rows_v, row1_v, out1_v, out2_v, rid_v, cmap_v, sem):
    wid = lax.axis_index("s") * _NC + lax.axis_index("c")

    def one_phase(nn, per, idx_hbm, out_hbm, out_v):
      b = (wid * per) // nn
      p0 = (wid * per) % nn
      pltpu.sync_copy(idx_hbm.at[b, :], cmap_v.at[pl.ds(0, nn)])
      for c0 in range(0, per, ch):
        pltpu.sync_copy(idx_hbm.at[b, pl.ds(p0 + c0, ch)], rid_v)
        _vaddupdate_offset(rid_v, ch, b * N)
        pltpu.async_copy(ls_ref.at[rid_v], rows_v, sem).wait()

        def row_body(r, carry):
          rsp = jnp.full((16,), r, jnp.int32)
          for qc in range(nn // 16):
            sl = pl.ds(qc * 16, 16)
            cidx = cmap_v[sl]
            out_v[r, sl] = plsc.load_gather(rows_v, [rsp, cidx])
          return carry

        lax.fori_loop(0, ch, row_body, 0)
        pltpu.sync_copy(out_v,
                        out_hbm.at[pl.ds(wid * per + c0, ch)])

    one_phase(n1, rows1_per, i0_ref, o1_ref, out1_v)
    one_phase(n2, rows2_per, sel_ref, o2_ref, out2_v)

  return kern(ls_hbm, i0_hbm, sel_hbm)


def _sc_gather_sum(table, nbt, k, n_out, trans=None, trans_n=0, self_off=0):
  mesh = plsc.VectorSubcoreMesh(core_axis_name="c", subcore_axis_name="s", num_cores=_NC)
  per = n_out // _NW
  ch = min(per, 128)

  scratch = [
      pltpu.VMEM((ch, H), jnp.float32),
      pltpu.VMEM((k, ch, H), jnp.float32),
      pltpu.VMEM((k, ch), jnp.int32),
      pltpu.VMEM((ch,), jnp.int32),
      pltpu.SemaphoreType.DMA,
  ]
  if trans is not None:
    scratch.append(pltpu.VMEM((trans_n,), jnp.int32))

  @functools.partial(
      pl.kernel,
      out_type=jax.ShapeDtypeStruct((n_out, H), jnp.float32),
      mesh=mesh,
      compiler_params=pltpu.CompilerParams(needs_layout_passes=False, use_tc_tiling_on_sc=False),
      scratch_types=scratch,
  )
  def kern(table_ref, nbt_ref, *rest):
    if trans is not None:
      trans_ref = rest[0]
      out_ref, acc_v, buf_v, ids_v, stage_v, sem, tmap_v = rest[1:]
    else:
      trans_ref = None
      out_ref, acc_v, buf_v, ids_v, stage_v, sem = rest

    wid = lax.axis_index("s") * _NC + lax.axis_index("c")
    base = wid * per
    if trans is not None:
      nb = n_out // B
      b = (wid * per) // nb
      pltpu.sync_copy(trans_ref.at[b, :], tmap_v)

    for c0 in range(0, per, ch):
      start = base + c0
      if trans is None:
        pltpu.sync_copy(table_ref.at[pl.ds(start, ch)], acc_v)
      else:
        p0 = (wid * per) % nb + c0
        pltpu.sync_copy(trans_ref.at[b, pl.ds(p0, ch)], stage_v)
        _vaddupdate_offset(stage_v, ch, b * self_off)
        pltpu.async_copy(table_ref.at[stage_v], acc_v, sem).wait()
      for j in range(k):
        if trans is None:
          pltpu.sync_copy(nbt_ref.at[j, pl.ds(start, ch)], ids_v.at[j])
        else:
          pltpu.sync_copy(nbt_ref.at[j, pl.ds(start, ch)], stage_v)

          def tr_body(cc, carry, j=j):
            sl = pl.ds(cc * 16, 16)
            q = stage_v[sl]
            ids_v[j, sl] = plsc.load_gather(tmap_v, [q]) + b * self_off
            return carry

          lax.fori_loop(0, ch // 16, tr_body, 0)
      copies = [pltpu.async_copy(table_ref.at[ids_v.at[j]], buf_v.at[j], sem)
                for j in range(k)]
      for cp in copies:
        cp.wait()
      for j in range(k):
        _vadd(acc_v, buf_v.at[j], ch, H)
      pltpu.sync_copy(acc_v, out_ref.at[pl.ds(start, ch)])

  if trans is not None:
    return kern(table, nbt, trans)
  return kern(table, nbt)


def kernel(x, edge_score, node_score, fc1_W, fc1_b, fc2_W, fc2_b,
           gnn_W0, gnn_b0, gnn_W1, gnn_b1, gnn_W2, gnn_b2):
  n1, n2 = N // DSN, N // (DSN * DSN)
  k0, k1, k2 = K0, K0 // DSE, K0 // (DSE * DSE)

  key = jax.random.key(42)

  def gumbel(k_, shape):
    u = jax.random.uniform(k_, shape, minval=1e-20, maxval=1.0)
    return -jnp.log(-jnp.log(u))

  g_e0 = gumbel(jax.random.fold_in(key, 0), (B * N, N))
  g_e1 = gumbel(jax.random.fold_in(key, 2), (B * n1, n1))
  g_e2 = gumbel(jax.random.fold_in(key, 4), (B * n2, n2))
  g_n0 = gumbel(jax.random.fold_in(key, 1), (B, N))
  g_n1 = gumbel(jax.random.fold_in(key, 3), (B, n1))

  eye = jnp.eye(N, dtype=bool)[None, :, :]
  es = jax.nn.softmax(jnp.where(eye, -jnp.inf, -edge_score), axis=-1)
  ls = jnp.where(es > 0, jnp.log(jnp.maximum(es, 1e-30)), -jnp.inf)
  ns = jax.nn.softmax(-node_score, axis=-1)
  ln = jnp.where(ns > 0, jnp.log(jnp.maximum(ns, 1e-30)), -jnp.inf)

  i0, i1, sel = _node_select(
      ln.reshape(B, 8, 128),
      g_n0.reshape(B, 8, 128),
      g_n1.reshape(B, 4, 128),
  )

  ls2 = ls.reshape(B * N, N)
  q0 = _edge_topk(ls2, g_e0, k0, N, add_batch_off=True)
  lsg1, lsg2 = _sc_gather_scores(ls2, i0, sel, n1, n2)
  q1 = _edge_topk(lsg1, g_e1, k1, n1, add_batch_off=False)
  q2 = _edge_topk(lsg2, g_e2, k2, n2, add_batch_off=False)

  q0t = q0.T.reshape(k0, B * N)
  q1t = q1.T.reshape(k1, B * n1)
  q2t = q2.T.reshape(k2, B * n2)

  xw0 = _sort_fc1(x.reshape(B * N, S), fc1_W, fc1_b.reshape(1, H), gnn_W0)
  sum0 = _sc_gather_sum(xw0, q0t, k0, B * N)
  xw1 = _scale_mm(sum0, gnn_b0.reshape(1, H), gnn_W1, 1.0 / 9.0)
  sum1 = _sc_gather_sum(xw1, q1t, k1, B * n1,
                        trans=i0, trans_n=n1, self_off=N)
  xw2 = _scale_mm(sum1, gnn_b1.reshape(1, H), gnn_W2, 1.0 / 5.0)
  sum2 = _sc_gather_sum(xw2, q2t, k2, B * n2,
                        trans=i1, trans_n=n2, self_off=n1)

  out = _final(sum2.reshape(B, n2, H), gnn_b2.reshape(1, H),
               fc2_W, fc2_b.reshape(1, 1))
  return out, sel


# speedup vs baseline: 14.3173x; 1.0711x over previous
import functools

import jax
import jax.numpy as jnp
from jax import lax
from jax.experimental import pallas as pl
from jax.experimental.pallas import tpu as pltpu
from jax.experimental.pallas import tpu_sc as plsc

B, N, S, H = 8, 1024, 128, 64
L, K0, DSN, DSE = 3, 8, 2, 2

_HIGH = jax.lax.Precision.HIGHEST
_NEG_INF = -float("inf")


def _topk_cols(s, kk, off):
  rows, width = s.shape
  li = lax.broadcasted_iota(jnp.int32, (rows, width), 1)
  big = jnp.int32(1 << 30)
  cols = []
  for _ in range(kk):
    m = jnp.max(s, axis=1, keepdims=True)
    cand = jnp.where(s == m, li, big)
    a = jnp.min(cand, axis=1, keepdims=True)
    cols.append(a + off)
    s = jnp.where(li == a, _NEG_INF, s)
  return jnp.concatenate(cols, axis=1)


def _bitonic_desc(key, payloads, nseg):
  n = nseg * 128
  shape = key.shape
  pos = (lax.broadcasted_iota(jnp.int32, shape, 1) * 128
         + lax.broadcasted_iota(jnp.int32, shape, 2))
  lane = lax.broadcasted_iota(jnp.int32, shape, 2)
  k = 2
  while k <= n:
    j = k // 2
    while j >= 1:
      if j >= 128:
        sj = j // 128
        perm = [seg ^ sj for seg in range(nseg)]
        pk = jnp.concatenate([key[:, q:q + 1, :] for q in perm], axis=1)
        pp = [jnp.concatenate([p[:, q:q + 1, :] for q in perm], axis=1)
              for p in payloads]
      else:
        gidx = lane ^ j
        pk = jnp.take_along_axis(key, gidx, axis=2)
        pp = [jnp.take_along_axis(p, gidx, axis=2) for p in payloads]
      w = (key > pk) | ((key == pk) & (payloads[0] < pp[0]))
      tw = ((pos & j) == 0) == ((pos & k) == 0)
      keep = w == tw
      key = jnp.where(keep, key, pk)
      payloads = [jnp.where(keep, p, q) for p, q in zip(payloads, pp)]
      j //= 2
    k *= 2
  return key, payloads


def _bitonic_asc_vals(x):
  rows, n = x.shape
  lane = lax.broadcasted_iota(jnp.int32, (rows, n), 1)
  k = 2
  while k <= n:
    j = k // 2
    while j >= 1:
      px = jnp.take_along_axis(x, lane ^ j, axis=1)
      tw = ((lane & j) == 0) == ((lane & k) == 0)
      x = jnp.where(tw, jnp.minimum(x, px), jnp.maximum(x, px))
      j //= 2
    k *= 2
  return x


def _node_select_body(ln_ref, gn0_ref, gn1_ref, i0_ref, i1_ref, sel_ref):
  ln = ln_ref[...]
  key0 = ln + gn0_ref[...]
  idx = (lax.broadcasted_iota(jnp.int32, key0.shape, 1) * 128
         + lax.broadcasted_iota(jnp.int32, key0.shape, 2))
  _, (idx_s, ln_s) = _bitonic_desc(key0, [idx, ln], 8)
  i0 = idx_s[:, :4, :]
  ln1 = ln_s[:, :4, :]
  key1 = ln1 + gn1_ref[...]
  idx2 = (lax.broadcasted_iota(jnp.int32, key1.shape, 1) * 128
          + lax.broadcasted_iota(jnp.int32, key1.shape, 2))
  _, (idx2_s, sel_s) = _bitonic_desc(key1, [idx2, i0], 4)
  i0_ref[...] = i0
  i1_ref[...] = idx2_s[:, :2, :]
  sel_ref[...] = sel_s[:, :2, :]


def _node_select(ln3, gn03, gn13):
  out = pl.pallas_call(
      _node_select_body,
      out_shape=(
          jax.ShapeDtypeStruct((B, 4, 128), jnp.int32),
          jax.ShapeDtypeStruct((B, 2, 128), jnp.int32),
          jax.ShapeDtypeStruct((B, 2, 128), jnp.int32),
      ),
  )(ln3, gn03, gn13)
  i0 = out[0].reshape(B, N // 2)
  i1 = out[1].reshape(B, N // 4)
  sel = out[2].reshape(B, N // 4)
  return i0, i1, sel


def _edge0_body(blk_rows, es_ref, u_ref, q_ref, ls_ref):
  pid = pl.program_id(0)
  z = -es_ref[...]
  r = lax.broadcasted_iota(jnp.int32, z.shape, 0) + pid * blk_rows
  c = lax.broadcasted_iota(jnp.int32, z.shape, 1)
  z = jnp.where((r % N) == c, _NEG_INF, z)
  m = jnp.max(z, axis=1, keepdims=True)
  e = jnp.exp(z - m)
  p = e / jnp.sum(e, axis=1, keepdims=True)
  ls = jnp.where(p > 0, jnp.log(jnp.maximum(p, 1e-30)), _NEG_INF)
  ls_ref[...] = ls
  s = ls + (-jnp.log(-jnp.log(u_ref[...])))
  off = (pid * blk_rows // N) * N
  q_ref[...] = _topk_cols(s, K0, off)


def _edge0_topk(es2, u2, blk_rows=128):
  total, width = es2.shape
  body = functools.partial(_edge0_body, blk_rows)
  return pl.pallas_call(
      body,
      grid=(total // blk_rows,),
      in_specs=[
          pl.BlockSpec((blk_rows, width), lambda i: (i, 0)),
          pl.BlockSpec((blk_rows, width), lambda i: (i, 0)),
      ],
      out_specs=[
          pl.BlockSpec((blk_rows, K0), lambda i: (i, 0)),
          pl.BlockSpec((blk_rows, width), lambda i: (i, 0)),
      ],
      out_shape=[
          jax.ShapeDtypeStruct((total, K0), jnp.int32),
          jax.ShapeDtypeStruct((total, width), jnp.float32),
      ],
  )(es2, u2)


def _edge_topk_body(kk, ls_ref, u_ref, o_ref):
  s = ls_ref[...] + (-jnp.log(-jnp.log(u_ref[...])))
  o_ref[...] = _topk_cols(s, kk, 0)


def _edge_topk(lsg, u2, kk, blk_rows=128):
  total, width = lsg.shape
  body = functools.partial(_edge_topk_body, kk)
  return pl.pallas_call(
      body,
      grid=(total // blk_rows,),
      in_specs=[
          pl.BlockSpec((blk_rows, width), lambda i: (i, 0)),
          pl.BlockSpec((blk_rows, width), lambda i: (i, 0)),
      ],
      out_specs=pl.BlockSpec((blk_rows, kk), lambda i: (i, 0)),
      out_shape=jax.ShapeDtypeStruct((total, kk), jnp.int32),
  )(lsg, u2)


def _sort_fc1_body(x_ref, w1_ref, b1_ref, w0_ref, o_ref):
  xs = _bitonic_asc_vals(x_ref[...])
  h = lax.dot_general(xs, w1_ref[...], (((1,), (1,)), ((), ())),
                      precision=_HIGH) + b1_ref[...]
  o_ref[...] = lax.dot_general(h, w0_ref[...], (((1,), (1,)), ((), ())),
                               precision=_HIGH)


def _sort_fc1(x2, fc1_w, fc1_b, w0):
  blk = 512
  return pl.pallas_call(
      _sort_fc1_body,
      grid=(x2.shape[0] // blk,),
      in_specs=[
          pl.BlockSpec((blk, S), lambda i: (i, 0)),
          pl.BlockSpec((H, S), lambda i: (0, 0)),
          pl.BlockSpec((1, H), lambda i: (0, 0)),
          pl.BlockSpec((H, H), lambda i: (0, 0)),
      ],
      out_specs=pl.BlockSpec((blk, H), lambda i: (i, 0)),
      out_shape=jax.ShapeDtypeStruct((x2.shape[0], H), jnp.float32),
  )(x2, fc1_w, fc1_b, w0)


def _scale_mm_body(inv_deg, s_ref, b_ref, w_ref, o_ref):
  v = s_ref[...] * inv_deg + b_ref[...]
  xl = jnp.where(v >= 0, v, 0.01 * v)
  o_ref[...] = lax.dot_general(xl, w_ref[...], (((1,), (1,)), ((), ())),
                               precision=_HIGH)


def _scale_mm(sums, bias, w, inv_deg):
  blk = 512
  body = functools.partial(_scale_mm_body, inv_deg)
  return pl.pallas_call(
      body,
      grid=(sums.shape[0] // blk,),
      in_specs=[
          pl.BlockSpec((blk, H), lambda i: (i, 0)),
          pl.BlockSpec((1, H), lambda i: (0, 0)),
          pl.BlockSpec((H, H), lambda i: (0, 0)),
      ],
      out_specs=pl.BlockSpec((blk, H), lambda i: (i, 0)),
      out_shape=jax.ShapeDtypeStruct((sums.shape[0], H), jnp.float32),
  )(sums, bias, w)


def _final_body(s_ref, b2_ref, fw_ref, fb_ref, o_ref):
  v = s_ref[0] * (1.0 / 3.0) + b2_ref[...]
  x3 = jnp.where(v >= 0, v, 0.01 * v)
  fw = fw_ref[...]
  y_col = lax.dot_general(x3, fw, (((1,), (1,)), ((), ())),
                          precision=_HIGH)
  nf = y_col.shape[0]
  ones = jnp.full((nf, 1), 1.0, jnp.float32)
  cdims = (((1,), (1,)), ((), ()))
  d = (lax.dot_general(y_col, ones, cdims, precision=_HIGH)
       - lax.dot_general(ones, y_col, cdims, precision=_HIGH))
  o_ref[0] = 1.0 / (1.0 + jnp.exp(-d))


def _final(sum2_3d, b2, fc2_w, fc2_b):
  nf = sum2_3d.shape[1]
  return pl.pallas_call(
      _final_body,
      grid=(B,),
      in_specs=[
          pl.BlockSpec((1, nf, H), lambda i: (i, 0, 0)),
          pl.BlockSpec((1, H), lambda i: (0, 0)),
          pl.BlockSpec((1, H), lambda i: (0, 0)),
          pl.BlockSpec((1, 1), lambda i: (0, 0)),
      ],
      out_specs=pl.BlockSpec((1, nf, nf), lambda i: (i, 0, 0)),
      out_shape=jax.ShapeDtypeStruct((B, nf, nf), jnp.float32),
  )(sum2_3d, b2, fc2_w, fc2_b)


_NC = 2
_NS = 16
_NW = _NC * _NS


def _vadd(acc_ref, buf_ref, rows, cols):
  nv = cols // 16

  def body(r, c):
    for cc in range(nv):
      sl = pl.ds(cc * 16, 16)
      acc_ref[r, sl] = acc_ref[r, sl] + buf_ref[r, sl]
    return c

  lax.fori_loop(0, rows, body, 0)


def _vaddupdate_offset(idx_ref, n, off):
  def body(c, carry):
    sl = pl.ds(c * 16, 16)
    idx_ref[sl] = idx_ref[sl] + off
    return carry

  lax.fori_loop(0, n // 16, body, 0)


def _sc_gather_scores(ls_hbm, i0_hbm, sel_hbm, n1, n2):
  mesh = plsc.VectorSubcoreMesh(core_axis_name="c", subcore_axis_name="s", num_cores=_NC)
  rows1_per = B * n1 // _NW
  rows2_per = B * n2 // _NW
  ch = 64

  @functools.partial(
      pl.kernel,
      out_type=(
          jax.ShapeDtypeStruct((B * n1, n1), jnp.float32),
          jax.ShapeDtypeStruct((B * n2, n2), jnp.float32),
      ),
      mesh=mesh,
      compiler_params=pltpu.CompilerParams(needs_layout_passes=False, use_tc_tiling_on_sc=False),
      scratch_types=[
          pltpu.VMEM((ch, N), jnp.float32),
          pltpu.VMEM((N,), jnp.float32),
          pltpu.VMEM((ch, n1), jnp.float32),
          pltpu.VMEM((ch, n2), jnp.float32),
          pltpu.VMEM((ch,), jnp.int32),
          pltpu.VMEM((n1,), jnp.int32),
          pltpu.SemaphoreType.DMA,
      ],
  )
  def kern(ls_ref, i0_ref, sel_ref, o1_ref, o2_ref,
           rows_v, row1_v, out1_v, out2_v, rid_v, cmap_v, sem):
    wid = lax.axis_index("s") * _NC + lax.axis_index("c")

    def one_phase(nn, per, idx_hbm, out_hbm, out_v):
      b = (wid * per) // nn
      p0 = (wid * per) % nn
      pltpu.sync_copy(idx_hbm.at[b, :], cmap_v.at[pl.ds(0, nn)])
      for c0 in range(0, per, ch):
        pltpu.sync_copy(idx_hbm.at[b, pl.ds(p0 + c0, ch)], rid_v)
        _vaddupdate_offset(rid_v, ch, b * N)
        pltpu.async_copy(ls_ref.at[rid_v], rows_v, sem).wait()

        def row_body(r, carry):
          rsp = jnp.full((16,), r, jnp.int32)
          for qc in range(nn // 16):
            sl = pl.ds(qc * 16, 16)
            cidx = cmap_v[sl]
            out_v[r, sl] = plsc.load_gather(rows_v, [rsp, cidx])
          return carry

        lax.fori_loop(0, ch, row_body, 0)
        pltpu.sync_copy(out_v,
                        out_hbm.at[pl.ds(wid * per + c0, ch)])

    one_phase(n1, rows1_per, i0_ref, o1_ref, out1_v)
    one_phase(n2, rows2_per, sel_ref, o2_ref, out2_v)

  return kern(ls_hbm, i0_hbm, sel_hbm)


def _sc_gather_sum(table, nbt, k, n_out, trans=None, trans_n=0, self_off=0):
  mesh = plsc.VectorSubcoreMesh(core_axis_name="c", subcore_axis_name="s", num_cores=_NC)
  per = n_out // _NW
  ch = min(per, 128)

  scratch = [
      pltpu.VMEM((ch, H), jnp.float32),
      pltpu.VMEM((k, ch, H), jnp.float32),
      pltpu.VMEM((k, ch), jnp.int32),
      pltpu.VMEM((ch,), jnp.int32),
      pltpu.SemaphoreType.DMA,
  ]
  if trans is not None:
    scratch.append(pltpu.VMEM((trans_n,), jnp.int32))

  @functools.partial(
      pl.kernel,
      out_type=jax.ShapeDtypeStruct((n_out, H), jnp.float32),
      mesh=mesh,
      compiler_params=pltpu.CompilerParams(needs_layout_passes=False, use_tc_tiling_on_sc=False),
      scratch_types=scratch,
  )
  def kern(table_ref, nbt_ref, *rest):
    if trans is not None:
      trans_ref = rest[0]
      out_ref, acc_v, buf_v, ids_v, stage_v, sem, tmap_v = rest[1:]
    else:
      trans_ref = None
      out_ref, acc_v, buf_v, ids_v, stage_v, sem = rest

    wid = lax.axis_index("s") * _NC + lax.axis_index("c")
    base = wid * per
    if trans is not None:
      nb = n_out // B
      b = (wid * per) // nb
      pltpu.sync_copy(trans_ref.at[b, :], tmap_v)

    for c0 in range(0, per, ch):
      start = base + c0
      if trans is None:
        pltpu.sync_copy(table_ref.at[pl.ds(start, ch)], acc_v)
      else:
        p0 = (wid * per) % nb + c0
        pltpu.sync_copy(trans_ref.at[b, pl.ds(p0, ch)], stage_v)
        _vaddupdate_offset(stage_v, ch, b * self_off)
        pltpu.async_copy(table_ref.at[stage_v], acc_v, sem).wait()
      for j in range(k):
        if trans is None:
          pltpu.sync_copy(nbt_ref.at[j, pl.ds(start, ch)], ids_v.at[j])
        else:
          pltpu.sync_copy(nbt_ref.at[j, pl.ds(start, ch)], stage_v)

          def tr_body(cc, carry, j=j):
            sl = pl.ds(cc * 16, 16)
            q = stage_v[sl]
            ids_v[j, sl] = plsc.load_gather(tmap_v, [q]) + b * self_off
            return carry

          lax.fori_loop(0, ch // 16, tr_body, 0)
      copies = [pltpu.async_copy(table_ref.at[ids_v.at[j]], buf_v.at[j], sem)
                for j in range(k)]
      for cp in copies:
        cp.wait()
      for j in range(k):
        _vadd(acc_v, buf_v.at[j], ch, H)
      pltpu.sync_copy(acc_v, out_ref.at[pl.ds(start, ch)])

  if trans is not None:
    return kern(table, nbt, trans)
  return kern(table, nbt)


def kernel(x, edge_score, node_score, fc1_W, fc1_b, fc2_W, fc2_b,
           gnn_W0, gnn_b0, gnn_W1, gnn_b1, gnn_W2, gnn_b2):
  n1, n2 = N // DSN, N // (DSN * DSN)
  k0, k1, k2 = K0, K0 // DSE, K0 // (DSE * DSE)

  key = jax.random.key(42)

  def unif(k_, shape):
    return jax.random.uniform(k_, shape, minval=1e-20, maxval=1.0)

  def gumbel(k_, shape):
    return -jnp.log(-jnp.log(unif(k_, shape)))

  u_e0 = unif(jax.random.fold_in(key, 0), (B * N, N))
  u_e1 = unif(jax.random.fold_in(key, 2), (B * n1, n1))
  u_e2 = unif(jax.random.fold_in(key, 4), (B * n2, n2))
  g_n0 = gumbel(jax.random.fold_in(key, 1), (B, N))
  g_n1 = gumbel(jax.random.fold_in(key, 3), (B, n1))

  ns = jax.nn.softmax(-node_score, axis=-1)
  ln = jnp.where(ns > 0, jnp.log(jnp.maximum(ns, 1e-30)), -jnp.inf)

  i0, i1, sel = _node_select(
      ln.reshape(B, 8, 128),
      g_n0.reshape(B, 8, 128),
      g_n1.reshape(B, 4, 128),
  )

  q0, ls2 = _edge0_topk(edge_score.reshape(B * N, N), u_e0)
  lsg1, lsg2 = _sc_gather_scores(ls2, i0, sel, n1, n2)
  q1 = _edge_topk(lsg1, u_e1, k1)
  q2 = _edge_topk(lsg2, u_e2, k2)

  q0t = q0.T.reshape(k0, B * N)
  q1t = q1.T.reshape(k1, B * n1)
  q2t = q2.T.reshape(k2, B * n2)

  xw0 = _sort_fc1(x.reshape(B * N, S), fc1_W, fc1_b.reshape(1, H), gnn_W0)
  sum0 = _sc_gather_sum(xw0, q0t, k0, B * N)
  xw1 = _scale_mm(sum0, gnn_b0.reshape(1, H), gnn_W1, 1.0 / 9.0)
  sum1 = _sc_gather_sum(xw1, q1t, k1, B * n1,
                        trans=i0, trans_n=n1, self_off=N)
  xw2 = _scale_mm(sum1, gnn_b1.reshape(1, H), gnn_W2, 1.0 / 5.0)
  sum2 = _sc_gather_sum(xw2, q2t, k2, B * n2,
                        trans=i1, trans_n=n2, self_off=n1)

  out = _final(sum2.reshape(B, n2, H), gnn_b2.reshape(1, H),
               fc2_W, fc2_b.reshape(1, 1))
  return out, sel


# speedup vs baseline: 14.4337x; 1.0081x over previous
import functools

import jax
import jax.numpy as jnp
from jax import lax
from jax.experimental import pallas as pl
from jax.experimental.pallas import tpu as pltpu
from jax.experimental.pallas import tpu_sc as plsc

B, N, S, H = 8, 1024, 128, 64
L, K0, DSN, DSE = 3, 8, 2, 2

_HIGH = jax.lax.Precision.HIGHEST
_NEG_INF = -float("inf")


def _topk_cols(s, kk, off):
  rows, width = s.shape
  li = lax.broadcasted_iota(jnp.int32, (rows, width), 1)
  big = jnp.int32(1 << 30)
  cols = []
  for _ in range(kk):
    m = jnp.max(s, axis=1, keepdims=True)
    cand = jnp.where(s == m, li, big)
    a = jnp.min(cand, axis=1, keepdims=True)
    cols.append(a + off)
    s = jnp.where(li == a, _NEG_INF, s)
  return jnp.concatenate(cols, axis=1)


def _bitonic_desc(key, payloads, nseg):
  n = nseg * 128
  shape = key.shape
  pos = (lax.broadcasted_iota(jnp.int32, shape, 1) * 128
         + lax.broadcasted_iota(jnp.int32, shape, 2))
  lane = lax.broadcasted_iota(jnp.int32, shape, 2)
  k = 2
  while k <= n:
    j = k // 2
    while j >= 1:
      if j >= 128:
        sj = j // 128
        perm = [seg ^ sj for seg in range(nseg)]
        pk = jnp.concatenate([key[:, q:q + 1, :] for q in perm], axis=1)
        pp = [jnp.concatenate([p[:, q:q + 1, :] for q in perm], axis=1)
              for p in payloads]
      else:
        gidx = lane ^ j
        pk = jnp.take_along_axis(key, gidx, axis=2)
        pp = [jnp.take_along_axis(p, gidx, axis=2) for p in payloads]
      w = (key > pk) | ((key == pk) & (payloads[0] < pp[0]))
      tw = ((pos & j) == 0) == ((pos & k) == 0)
      keep = w == tw
      key = jnp.where(keep, key, pk)
      payloads = [jnp.where(keep, p, q) for p, q in zip(payloads, pp)]
      j //= 2
    k *= 2
  return key, payloads


def _bitonic_asc_vals(x):
  rows, n = x.shape
  lane = lax.broadcasted_iota(jnp.int32, (rows, n), 1)
  k = 2
  while k <= n:
    j = k // 2
    while j >= 1:
      px = jnp.take_along_axis(x, lane ^ j, axis=1)
      tw = ((lane & j) == 0) == ((lane & k) == 0)
      x = jnp.where(tw, jnp.minimum(x, px), jnp.maximum(x, px))
      j //= 2
    k *= 2
  return x


def _node_select_body(ln_ref, gn0_ref, gn1_ref, i0_ref, i1_ref, sel_ref):
  ln = ln_ref[...]
  key0 = ln + gn0_ref[...]
  idx = (lax.broadcasted_iota(jnp.int32, key0.shape, 1) * 128
         + lax.broadcasted_iota(jnp.int32, key0.shape, 2))
  _, (idx_s, ln_s) = _bitonic_desc(key0, [idx, ln], 8)
  i0 = idx_s[:, :4, :]
  ln1 = ln_s[:, :4, :]
  key1 = ln1 + gn1_ref[...]
  idx2 = (lax.broadcasted_iota(jnp.int32, key1.shape, 1) * 128
          + lax.broadcasted_iota(jnp.int32, key1.shape, 2))
  _, (idx2_s, sel_s) = _bitonic_desc(key1, [idx2, i0], 4)
  i0_ref[...] = i0
  i1_ref[...] = idx2_s[:, :2, :]
  sel_ref[...] = sel_s[:, :2, :]


def _node_select(ln3, gn03, gn13):
  out = pl.pallas_call(
      _node_select_body,
      out_shape=(
          jax.ShapeDtypeStruct((B, 4, 128), jnp.int32),
          jax.ShapeDtypeStruct((B, 2, 128), jnp.int32),
          jax.ShapeDtypeStruct((B, 2, 128), jnp.int32),
      ),
  )(ln3, gn03, gn13)
  i0 = out[0].reshape(B, N // 2)
  i1 = out[1].reshape(B, N // 4)
  sel = out[2].reshape(B, N // 4)
  return i0, i1, sel


def _edge0_body(blk_rows, es_ref, u_ref, q_ref, ls_ref):
  pid = pl.program_id(0)
  z = -es_ref[...]
  r = lax.broadcasted_iota(jnp.int32, z.shape, 0) + pid * blk_rows
  c = lax.broadcasted_iota(jnp.int32, z.shape, 1)
  z = jnp.where((r % N) == c, _NEG_INF, z)
  m = jnp.max(z, axis=1, keepdims=True)
  e = jnp.exp(z - m)
  p = e / jnp.sum(e, axis=1, keepdims=True)
  ls = jnp.where(p > 0, jnp.log(jnp.maximum(p, 1e-30)), _NEG_INF)
  ls_ref[...] = ls
  s = ls + (-jnp.log(-jnp.log(u_ref[...])))
  off = (pid * blk_rows // N) * N
  q_ref[...] = _topk_cols(s, K0, off)


def _edge0_topk(es2, u2, blk_rows=128):
  total, width = es2.shape
  body = functools.partial(_edge0_body, blk_rows)
  return pl.pallas_call(
      body,
      grid=(total // blk_rows,),
      in_specs=[
          pl.BlockSpec((blk_rows, width), lambda i: (i, 0)),
          pl.BlockSpec((blk_rows, width), lambda i: (i, 0)),
      ],
      out_specs=[
          pl.BlockSpec((blk_rows, K0), lambda i: (i, 0)),
          pl.BlockSpec((blk_rows, width), lambda i: (i, 0)),
      ],
      out_shape=[
          jax.ShapeDtypeStruct((total, K0), jnp.int32),
          jax.ShapeDtypeStruct((total, width), jnp.float32),
      ],
  )(es2, u2)


def _edge_topk_body(kk, ls_ref, u_ref, o_ref):
  s = ls_ref[...] + (-jnp.log(-jnp.log(u_ref[...])))
  o_ref[...] = _topk_cols(s, kk, 0)


def _edge_topk(lsg, u2, kk, blk_rows=128):
  total, width = lsg.shape
  body = functools.partial(_edge_topk_body, kk)
  return pl.pallas_call(
      body,
      grid=(total // blk_rows,),
      in_specs=[
          pl.BlockSpec((blk_rows, width), lambda i: (i, 0)),
          pl.BlockSpec((blk_rows, width), lambda i: (i, 0)),
      ],
      out_specs=pl.BlockSpec((blk_rows, kk), lambda i: (i, 0)),
      out_shape=jax.ShapeDtypeStruct((total, kk), jnp.int32),
  )(lsg, u2)


def _sort_fc1_body(x_ref, w1_ref, b1_ref, w0_ref, o_ref):
  xs = _bitonic_asc_vals(x_ref[...])
  h = lax.dot_general(xs, w1_ref[...], (((1,), (1,)), ((), ())),
                      precision=_HIGH) + b1_ref[...]
  o_ref[...] = lax.dot_general(h, w0_ref[...], (((1,), (1,)), ((), ())),
                               precision=_HIGH)


def _sort_fc1(x2, fc1_w, fc1_b, w0):
  blk = 512
  return pl.pallas_call(
      _sort_fc1_body,
      grid=(x2.shape[0] // blk,),
      in_specs=[
          pl.BlockSpec((blk, S), lambda i: (i, 0)),
          pl.BlockSpec((H, S), lambda i: (0, 0)),
          pl.BlockSpec((1, H), lambda i: (0, 0)),
          pl.BlockSpec((H, H), lambda i: (0, 0)),
      ],
      out_specs=pl.BlockSpec((blk, H), lambda i: (i, 0)),
      out_shape=jax.ShapeDtypeStruct((x2.shape[0], H), jnp.float32),
  )(x2, fc1_w, fc1_b, w0)


def _scale_mm_body(inv_deg, s_ref, b_ref, w_ref, o_ref):
  v = s_ref[...] * inv_deg + b_ref[...]
  xl = jnp.where(v >= 0, v, 0.01 * v)
  o_ref[...] = lax.dot_general(xl, w_ref[...], (((1,), (1,)), ((), ())),
                               precision=_HIGH)


def _scale_mm(sums, bias, w, inv_deg):
  blk = 512
  body = functools.partial(_scale_mm_body, inv_deg)
  return pl.pallas_call(
      body,
      grid=(sums.shape[0] // blk,),
      in_specs=[
          pl.BlockSpec((blk, H), lambda i: (i, 0)),
          pl.BlockSpec((1, H), lambda i: (0, 0)),
          pl.BlockSpec((H, H), lambda i: (0, 0)),
      ],
      out_specs=pl.BlockSpec((blk, H), lambda i: (i, 0)),
      out_shape=jax.ShapeDtypeStruct((sums.shape[0], H), jnp.float32),
  )(sums, bias, w)


def _final_body(s_ref, b2_ref, fw_ref, fb_ref, o_ref):
  v = s_ref[0] * (1.0 / 3.0) + b2_ref[...]
  x3 = jnp.where(v >= 0, v, 0.01 * v)
  fw = fw_ref[...]
  y_col = lax.dot_general(x3, fw, (((1,), (1,)), ((), ())),
                          precision=_HIGH)
  nf = y_col.shape[0]
  ones = jnp.full((nf, 1), 1.0, jnp.float32)
  cdims = (((1,), (1,)), ((), ()))
  d = (lax.dot_general(y_col, ones, cdims, precision=_HIGH)
       - lax.dot_general(ones, y_col, cdims, precision=_HIGH))
  o_ref[0] = 1.0 / (1.0 + jnp.exp(-d))


def _final(sum2_3d, b2, fc2_w, fc2_b):
  nf = sum2_3d.shape[1]
  return pl.pallas_call(
      _final_body,
      grid=(B,),
      in_specs=[
          pl.BlockSpec((1, nf, H), lambda i: (i, 0, 0)),
          pl.BlockSpec((1, H), lambda i: (0, 0)),
          pl.BlockSpec((1, H), lambda i: (0, 0)),
          pl.BlockSpec((1, 1), lambda i: (0, 0)),
      ],
      out_specs=pl.BlockSpec((1, nf, nf), lambda i: (i, 0, 0)),
      out_shape=jax.ShapeDtypeStruct((B, nf, nf), jnp.float32),
  )(sum2_3d, b2, fc2_w, fc2_b)


_NC = 2
_NS = 16
_NW = _NC * _NS


def _vadd(acc_ref, buf_ref, rows, cols):
  nv = cols // 16

  def body(r, c):
    for cc in range(nv):
      sl = pl.ds(cc * 16, 16)
      acc_ref[r, sl] = acc_ref[r, sl] + buf_ref[r, sl]
    return c

  lax.fori_loop(0, rows, body, 0)


def _vaddupdate_offset(idx_ref, n, off):
  def body(c, carry):
    sl = pl.ds(c * 16, 16)
    idx_ref[sl] = idx_ref[sl] + off
    return carry

  lax.fori_loop(0, n // 16, body, 0)


def _sc_gather_scores(ls_hbm, i0_hbm, sel_hbm, n1, n2):
  mesh = plsc.VectorSubcoreMesh(core_axis_name="c", subcore_axis_name="s", num_cores=_NC)
  rows1_per = B * n1 // _NW
  rows2_per = B * n2 // _NW
  ch = 32

  @functools.partial(
      pl.kernel,
      out_type=(
          jax.ShapeDtypeStruct((B * n1, n1), jnp.float32),
          jax.ShapeDtypeStruct((B * n2, n2), jnp.float32),
      ),
      mesh=mesh,
      compiler_params=pltpu.CompilerParams(needs_layout_passes=False, use_tc_tiling_on_sc=False),
      scratch_types=[
          pltpu.VMEM((2, ch, N), jnp.float32),
          pltpu.VMEM((ch, n1), jnp.float32),
          pltpu.VMEM((ch, n2), jnp.float32),
          pltpu.VMEM((2, ch), jnp.int32),
          pltpu.VMEM((n1,), jnp.int32),
          pltpu.SemaphoreType.DMA,
      ],
  )
  def kern(ls_ref, i0_ref, sel_ref, o1_ref, o2_ref,
           rows_v, out1_v, out2_v, rid_v, cmap_v, sem):
    wid = lax.axis_index("s") * _NC + lax.axis_index("c")

    def one_phase(nn, per, idx_hbm, out_hbm, out_v):
      b = (wid * per) // nn
      p0 = (wid * per) % nn
      nch = per // ch
      pltpu.sync_copy(idx_hbm.at[b, :], cmap_v.at[pl.ds(0, nn)])

      def start(c):
        buf = c % 2
        pltpu.sync_copy(idx_hbm.at[b, pl.ds(p0 + c * ch, ch)],
                        rid_v.at[buf])
        _vaddupdate_offset(rid_v.at[buf], ch, b * N)
        return pltpu.async_copy(ls_ref.at[rid_v.at[buf]], rows_v.at[buf],
                                sem)

      pending = start(0)
      for c in range(nch):
        nxt = start(c + 1) if c + 1 < nch else None
        pending.wait()
        buf = c % 2

        def row_body(r, carry, buf=buf):
          rsp = jnp.full((16,), r, jnp.int32)
          for qc in range(nn // 16):
            sl = pl.ds(qc * 16, 16)
            out_v[r, sl] = plsc.load_gather(rows_v.at[buf],
                                            [rsp, cmap_v[sl]])
          return carry

        lax.fori_loop(0, ch, row_body, 0)
        pltpu.sync_copy(out_v,
                        out_hbm.at[pl.ds(wid * per + c * ch, ch)])
        pending = nxt

    one_phase(n1, rows1_per, i0_ref, o1_ref, out1_v)
    one_phase(n2, rows2_per, sel_ref, o2_ref, out2_v)

  return kern(ls_hbm, i0_hbm, sel_hbm)


def _sc_gather_sum(table, nbt, k, n_out, trans=None, trans_n=0, self_off=0):
  mesh = plsc.VectorSubcoreMesh(core_axis_name="c", subcore_axis_name="s", num_cores=_NC)
  per = n_out // _NW
  ch = min(per, 128)

  scratch = [
      pltpu.VMEM((ch, H), jnp.float32),
      pltpu.VMEM((k, ch, H), jnp.float32),
      pltpu.VMEM((k, ch), jnp.int32),
      pltpu.VMEM((ch,), jnp.int32),
      pltpu.SemaphoreType.DMA,
  ]
  if trans is not None:
    scratch.append(pltpu.VMEM((trans_n,), jnp.int32))

  @functools.partial(
      pl.kernel,
      out_type=jax.ShapeDtypeStruct((n_out, H), jnp.float32),
      mesh=mesh,
      compiler_params=pltpu.CompilerParams(needs_layout_passes=False, use_tc_tiling_on_sc=False),
      scratch_types=scratch,
  )
  def kern(table_ref, nbt_ref, *rest):
    if trans is not None:
      trans_ref = rest[0]
      out_ref, acc_v, buf_v, ids_v, stage_v, sem, tmap_v = rest[1:]
    else:
      trans_ref = None
      out_ref, acc_v, buf_v, ids_v, stage_v, sem = rest

    wid = lax.axis_index("s") * _NC + lax.axis_index("c")
    base = wid * per
    if trans is not None:
      nb = n_out // B
      b = (wid * per) // nb
      pltpu.sync_copy(trans_ref.at[b, :], tmap_v)

    for c0 in range(0, per, ch):
      start = base + c0
      if trans is None:
        pltpu.sync_copy(table_ref.at[pl.ds(start, ch)], acc_v)
      else:
        p0 = (wid * per) % nb + c0
        pltpu.sync_copy(trans_ref.at[b, pl.ds(p0, ch)], stage_v)
        _vaddupdate_offset(stage_v, ch, b * self_off)
        pltpu.async_copy(table_ref.at[stage_v], acc_v, sem).wait()
      for j in range(k):
        if trans is None:
          pltpu.sync_copy(nbt_ref.at[j, pl.ds(start, ch)], ids_v.at[j])
        else:
          pltpu.sync_copy(nbt_ref.at[j, pl.ds(start, ch)], stage_v)

          def tr_body(cc, carry, j=j):
            sl = pl.ds(cc * 16, 16)
            q = stage_v[sl]
            ids_v[j, sl] = plsc.load_gather(tmap_v, [q]) + b * self_off
            return carry

          lax.fori_loop(0, ch // 16, tr_body, 0)
      copies = [pltpu.async_copy(table_ref.at[ids_v.at[j]], buf_v.at[j], sem)
                for j in range(k)]
      for cp in copies:
        cp.wait()
      for j in range(k):
        _vadd(acc_v, buf_v.at[j], ch, H)
      pltpu.sync_copy(acc_v, out_ref.at[pl.ds(start, ch)])

  if trans is not None:
    return kern(table, nbt, trans)
  return kern(table, nbt)


def kernel(x, edge_score, node_score, fc1_W, fc1_b, fc2_W, fc2_b,
           gnn_W0, gnn_b0, gnn_W1, gnn_b1, gnn_W2, gnn_b2):
  n1, n2 = N // DSN, N // (DSN * DSN)
  k0, k1, k2 = K0, K0 // DSE, K0 // (DSE * DSE)

  key = jax.random.key(42)

  def unif(k_, shape):
    return jax.random.uniform(k_, shape, minval=1e-20, maxval=1.0)

  def gumbel(k_, shape):
    return -jnp.log(-jnp.log(unif(k_, shape)))

  u_e0 = unif(jax.random.fold_in(key, 0), (B * N, N))
  u_e1 = unif(jax.random.fold_in(key, 2), (B * n1, n1))
  u_e2 = unif(jax.random.fold_in(key, 4), (B * n2, n2))
  g_n0 = gumbel(jax.random.fold_in(key, 1), (B, N))
  g_n1 = gumbel(jax.random.fold_in(key, 3), (B, n1))

  ns = jax.nn.softmax(-node_score, axis=-1)
  ln = jnp.where(ns > 0, jnp.log(jnp.maximum(ns, 1e-30)), -jnp.inf)

  i0, i1, sel = _node_select(
      ln.reshape(B, 8, 128),
      g_n0.reshape(B, 8, 128),
      g_n1.reshape(B, 4, 128),
  )

  q0, ls2 = _edge0_topk(edge_score.reshape(B * N, N), u_e0)
  lsg1, lsg2 = _sc_gather_scores(ls2, i0, sel, n1, n2)
  q1 = _edge_topk(lsg1, u_e1, k1)
  q2 = _edge_topk(lsg2, u_e2, k2)

  q0t = q0.T.reshape(k0, B * N)
  q1t = q1.T.reshape(k1, B * n1)
  q2t = q2.T.reshape(k2, B * n2)

  xw0 = _sort_fc1(x.reshape(B * N, S), fc1_W, fc1_b.reshape(1, H), gnn_W0)
  sum0 = _sc_gather_sum(xw0, q0t, k0, B * N)
  xw1 = _scale_mm(sum0, gnn_b0.reshape(1, H), gnn_W1, 1.0 / 9.0)
  sum1 = _sc_gather_sum(xw1, q1t, k1, B * n1,
                        trans=i0, trans_n=n1, self_off=N)
  xw2 = _scale_mm(sum1, gnn_b1.reshape(1, H), gnn_W2, 1.0 / 5.0)
  sum2 = _sc_gather_sum(xw2, q2t, k2, B * n2,
                        trans=i1, trans_n=n2, self_off=n1)

  out = _final(sum2.reshape(B, n2, H), gnn_b2.reshape(1, H),
               fc2_W, fc2_b.reshape(1, 1))
  return out, sel


# speedup vs baseline: 17.0955x; 1.1844x over previous
import functools

import jax
import jax.numpy as jnp
from jax import lax
from jax.experimental import pallas as pl
from jax.experimental.pallas import tpu as pltpu
from jax.experimental.pallas import tpu_sc as plsc

B, N, S, H = 8, 1024, 128, 64
L, K0, DSN, DSE = 3, 8, 2, 2

_HIGH = jax.lax.Precision.HIGHEST
_NEG_INF = -float("inf")


def _topk_cols(s, kk, off):
  rows, width = s.shape
  li = lax.broadcasted_iota(jnp.int32, (rows, width), 1)
  big = jnp.int32(1 << 30)
  cols = []
  for _ in range(kk):
    m = jnp.max(s, axis=1, keepdims=True)
    cand = jnp.where(s == m, li, big)
    a = jnp.min(cand, axis=1, keepdims=True)
    cols.append(a + off)
    s = jnp.where(li == a, _NEG_INF, s)
  return jnp.concatenate(cols, axis=1)


def _bitonic_desc(key, payloads, nseg):
  n = nseg * 128
  shape = key.shape
  pos = (lax.broadcasted_iota(jnp.int32, shape, 1) * 128
         + lax.broadcasted_iota(jnp.int32, shape, 2))
  lane = lax.broadcasted_iota(jnp.int32, shape, 2)
  k = 2
  while k <= n:
    j = k // 2
    while j >= 1:
      if j >= 128:
        sj = j // 128
        perm = [seg ^ sj for seg in range(nseg)]
        pk = jnp.concatenate([key[:, q:q + 1, :] for q in perm], axis=1)
        pp = [jnp.concatenate([p[:, q:q + 1, :] for q in perm], axis=1)
              for p in payloads]
      else:
        gidx = lane ^ j
        pk = jnp.take_along_axis(key, gidx, axis=2)
        pp = [jnp.take_along_axis(p, gidx, axis=2) for p in payloads]
      w = (key > pk) | ((key == pk) & (payloads[0] < pp[0]))
      tw = ((pos & j) == 0) == ((pos & k) == 0)
      keep = w == tw
      key = jnp.where(keep, key, pk)
      payloads = [jnp.where(keep, p, q) for p, q in zip(payloads, pp)]
      j //= 2
    k *= 2
  return key, payloads


def _bitonic_asc_vals(x):
  rows, n = x.shape
  lane = lax.broadcasted_iota(jnp.int32, (rows, n), 1)
  k = 2
  while k <= n:
    j = k // 2
    while j >= 1:
      px = jnp.take_along_axis(x, lane ^ j, axis=1)
      tw = ((lane & j) == 0) == ((lane & k) == 0)
      x = jnp.where(tw, jnp.minimum(x, px), jnp.maximum(x, px))
      j //= 2
    k *= 2
  return x


def _node_select_body(ln_ref, gn0_ref, gn1_ref, i0_ref, i1_ref, sel_ref):
  ln = ln_ref[...]
  key0 = ln + gn0_ref[...]
  idx = (lax.broadcasted_iota(jnp.int32, key0.shape, 1) * 128
         + lax.broadcasted_iota(jnp.int32, key0.shape, 2))
  _, (idx_s, ln_s) = _bitonic_desc(key0, [idx, ln], 8)
  i0 = idx_s[:, :4, :]
  ln1 = ln_s[:, :4, :]
  key1 = ln1 + gn1_ref[...]
  idx2 = (lax.broadcasted_iota(jnp.int32, key1.shape, 1) * 128
          + lax.broadcasted_iota(jnp.int32, key1.shape, 2))
  _, (idx2_s, sel_s) = _bitonic_desc(key1, [idx2, i0], 4)
  i0_ref[...] = i0
  i1_ref[...] = idx2_s[:, :2, :]
  sel_ref[...] = sel_s[:, :2, :]


def _node_select(ln3, gn03, gn13):
  out = pl.pallas_call(
      _node_select_body,
      out_shape=(
          jax.ShapeDtypeStruct((B, 4, 128), jnp.int32),
          jax.ShapeDtypeStruct((B, 2, 128), jnp.int32),
          jax.ShapeDtypeStruct((B, 2, 128), jnp.int32),
      ),
  )(ln3, gn03, gn13)
  i0 = out[0].reshape(B, N // 2)
  i1 = out[1].reshape(B, N // 4)
  sel = out[2].reshape(B, N // 4)
  return i0, i1, sel


def _edge0_body(blk_rows, es_ref, u_ref, q_ref, ls_ref):
  pid = pl.program_id(0)
  z = -es_ref[...]
  r = lax.broadcasted_iota(jnp.int32, z.shape, 0) + pid * blk_rows
  c = lax.broadcasted_iota(jnp.int32, z.shape, 1)
  z = jnp.where((r % N) == c, _NEG_INF, z)
  m = jnp.max(z, axis=1, keepdims=True)
  e = jnp.exp(z - m)
  p = e / jnp.sum(e, axis=1, keepdims=True)
  ls = jnp.where(p > 0, jnp.log(jnp.maximum(p, 1e-30)), _NEG_INF)
  ls_ref[...] = ls
  s = ls + (-jnp.log(-jnp.log(u_ref[...])))
  off = (pid * blk_rows // N) * N
  q_ref[...] = _topk_cols(s, K0, off)


def _edge0_topk(es2, u2, blk_rows=512):
  total, width = es2.shape
  body = functools.partial(_edge0_body, blk_rows)
  return pl.pallas_call(
      body,
      grid=(total // blk_rows,),
      in_specs=[
          pl.BlockSpec((blk_rows, width), lambda i: (i, 0)),
          pl.BlockSpec((blk_rows, width), lambda i: (i, 0)),
      ],
      out_specs=[
          pl.BlockSpec((blk_rows, K0), lambda i: (i, 0)),
          pl.BlockSpec((blk_rows, width), lambda i: (i, 0)),
      ],
      out_shape=[
          jax.ShapeDtypeStruct((total, K0), jnp.int32),
          jax.ShapeDtypeStruct((total, width), jnp.float32),
      ],
  )(es2, u2)


def _edge_topk_body(kk, ls_ref, u_ref, o_ref):
  s = ls_ref[...] + (-jnp.log(-jnp.log(u_ref[...])))
  o_ref[...] = _topk_cols(s, kk, 0)


def _edge_topk(lsg, u2, kk, blk_rows=512):
  total, width = lsg.shape
  body = functools.partial(_edge_topk_body, kk)
  return pl.pallas_call(
      body,
      grid=(total // blk_rows,),
      in_specs=[
          pl.BlockSpec((blk_rows, width), lambda i: (i, 0)),
          pl.BlockSpec((blk_rows, width), lambda i: (i, 0)),
      ],
      out_specs=pl.BlockSpec((blk_rows, kk), lambda i: (i, 0)),
      out_shape=jax.ShapeDtypeStruct((total, kk), jnp.int32),
  )(lsg, u2)


def _sort_fc1_body(x_ref, w1_ref, b1_ref, w0_ref, o_ref):
  xs = _bitonic_asc_vals(x_ref[...])
  h = lax.dot_general(xs, w1_ref[...], (((1,), (1,)), ((), ())),
                      precision=_HIGH) + b1_ref[...]
  o_ref[...] = lax.dot_general(h, w0_ref[...], (((1,), (1,)), ((), ())),
                               precision=_HIGH)


def _sort_fc1(x2, fc1_w, fc1_b, w0):
  blk = 512
  return pl.pallas_call(
      _sort_fc1_body,
      grid=(x2.shape[0] // blk,),
      in_specs=[
          pl.BlockSpec((blk, S), lambda i: (i, 0)),
          pl.BlockSpec((H, S), lambda i: (0, 0)),
          pl.BlockSpec((1, H), lambda i: (0, 0)),
          pl.BlockSpec((H, H), lambda i: (0, 0)),
      ],
      out_specs=pl.BlockSpec((blk, H), lambda i: (i, 0)),
      out_shape=jax.ShapeDtypeStruct((x2.shape[0], H), jnp.float32),
  )(x2, fc1_w, fc1_b, w0)


def _scale_mm_body(inv_deg, s_ref, b_ref, w_ref, o_ref):
  v = s_ref[...] * inv_deg + b_ref[...]
  xl = jnp.where(v >= 0, v, 0.01 * v)
  o_ref[...] = lax.dot_general(xl, w_ref[...], (((1,), (1,)), ((), ())),
                               precision=_HIGH)


def _scale_mm(sums, bias, w, inv_deg):
  blk = 1024
  body = functools.partial(_scale_mm_body, inv_deg)
  return pl.pallas_call(
      body,
      grid=(sums.shape[0] // blk,),
      in_specs=[
          pl.BlockSpec((blk, H), lambda i: (i, 0)),
          pl.BlockSpec((1, H), lambda i: (0, 0)),
          pl.BlockSpec((H, H), lambda i: (0, 0)),
      ],
      out_specs=pl.BlockSpec((blk, H), lambda i: (i, 0)),
      out_shape=jax.ShapeDtypeStruct((sums.shape[0], H), jnp.float32),
  )(sums, bias, w)


def _final_body(s_ref, b2_ref, fw_ref, fb_ref, o_ref):
  v = s_ref[0] * (1.0 / 3.0) + b2_ref[...]
  x3 = jnp.where(v >= 0, v, 0.01 * v)
  fw = fw_ref[...]
  y_col = lax.dot_general(x3, fw, (((1,), (1,)), ((), ())),
                          precision=_HIGH)
  nf = y_col.shape[0]
  ones = jnp.full((nf, 1), 1.0, jnp.float32)
  cdims = (((1,), (1,)), ((), ()))
  d = (lax.dot_general(y_col, ones, cdims, precision=_HIGH)
       - lax.dot_general(ones, y_col, cdims, precision=_HIGH))
  o_ref[0] = 1.0 / (1.0 + jnp.exp(-d))


def _final(sum2_3d, b2, fc2_w, fc2_b):
  nf = sum2_3d.shape[1]
  return pl.pallas_call(
      _final_body,
      grid=(B,),
      in_specs=[
          pl.BlockSpec((1, nf, H), lambda i: (i, 0, 0)),
          pl.BlockSpec((1, H), lambda i: (0, 0)),
          pl.BlockSpec((1, H), lambda i: (0, 0)),
          pl.BlockSpec((1, 1), lambda i: (0, 0)),
      ],
      out_specs=pl.BlockSpec((1, nf, nf), lambda i: (i, 0, 0)),
      out_shape=jax.ShapeDtypeStruct((B, nf, nf), jnp.float32),
  )(sum2_3d, b2, fc2_w, fc2_b)


_NC = 2
_NS = 16
_NW = _NC * _NS


def _vadd(acc_ref, buf_ref, rows, cols):
  nv = cols // 16

  def body(r, c):
    for cc in range(nv):
      sl = pl.ds(cc * 16, 16)
      acc_ref[r, sl] = acc_ref[r, sl] + buf_ref[r, sl]
    return c

  lax.fori_loop(0, rows, body, 0)


def _vaddupdate_offset(idx_ref, n, off):
  def body(c, carry):
    sl = pl.ds(c * 16, 16)
    idx_ref[sl] = idx_ref[sl] + off
    return carry

  lax.fori_loop(0, n // 16, body, 0)


def _sc_gather_scores(ls_hbm, i0_hbm, sel_hbm, n1, n2):
  mesh = plsc.VectorSubcoreMesh(core_axis_name="c", subcore_axis_name="s", num_cores=_NC)
  rows1_per = B * n1 // _NW
  rows2_per = B * n2 // _NW
  ch = 32

  @functools.partial(
      pl.kernel,
      out_type=(
          jax.ShapeDtypeStruct((B * n1, n1), jnp.float32),
          jax.ShapeDtypeStruct((B * n2, n2), jnp.float32),
      ),
      mesh=mesh,
      compiler_params=pltpu.CompilerParams(needs_layout_passes=False, use_tc_tiling_on_sc=False),
      scratch_types=[
          pltpu.VMEM((2, ch, N), jnp.float32),
          pltpu.VMEM((ch, n1), jnp.float32),
          pltpu.VMEM((ch, n2), jnp.float32),
          pltpu.VMEM((2, ch), jnp.int32),
          pltpu.VMEM((n1,), jnp.int32),
          pltpu.SemaphoreType.DMA,
      ],
  )
  def kern(ls_ref, i0_ref, sel_ref, o1_ref, o2_ref,
           rows_v, out1_v, out2_v, rid_v, cmap_v, sem):
    wid = lax.axis_index("s") * _NC + lax.axis_index("c")

    def one_phase(nn, per, idx_hbm, out_hbm, out_v):
      b = (wid * per) // nn
      p0 = (wid * per) % nn
      nch = per // ch
      pltpu.sync_copy(idx_hbm.at[b, :], cmap_v.at[pl.ds(0, nn)])

      def start(c):
        buf = c % 2
        pltpu.sync_copy(idx_hbm.at[b, pl.ds(p0 + c * ch, ch)],
                        rid_v.at[buf])
        _vaddupdate_offset(rid_v.at[buf], ch, b * N)
        return pltpu.async_copy(ls_ref.at[rid_v.at[buf]], rows_v.at[buf],
                                sem)

      pending = start(0)
      for c in range(nch):
        nxt = start(c + 1) if c + 1 < nch else None
        pending.wait()
        buf = c % 2

        def row_body(r, carry, buf=buf):
          rsp = jnp.full((16,), r, jnp.int32)
          for qc in range(nn // 16):
            sl = pl.ds(qc * 16, 16)
            out_v[r, sl] = plsc.load_gather(rows_v.at[buf],
                                            [rsp, cmap_v[sl]])
          return carry

        lax.fori_loop(0, ch, row_body, 0)
        pltpu.sync_copy(out_v,
                        out_hbm.at[pl.ds(wid * per + c * ch, ch)])
        pending = nxt

    one_phase(n1, rows1_per, i0_ref, o1_ref, out1_v)
    one_phase(n2, rows2_per, sel_ref, o2_ref, out2_v)

  return kern(ls_hbm, i0_hbm, sel_hbm)


def _sc_gather_sum(table, nbt, k, n_out, trans=None, trans_n=0, self_off=0):
  mesh = plsc.VectorSubcoreMesh(core_axis_name="c", subcore_axis_name="s", num_cores=_NC)
  per = n_out // _NW
  ch = min(per, 128)

  scratch = [
      pltpu.VMEM((ch, H), jnp.float32),
      pltpu.VMEM((k, ch, H), jnp.float32),
      pltpu.VMEM((k, ch), jnp.int32),
      pltpu.VMEM((ch,), jnp.int32),
      pltpu.SemaphoreType.DMA,
  ]
  if trans is not None:
    scratch.append(pltpu.VMEM((trans_n,), jnp.int32))

  @functools.partial(
      pl.kernel,
      out_type=jax.ShapeDtypeStruct((n_out, H), jnp.float32),
      mesh=mesh,
      compiler_params=pltpu.CompilerParams(needs_layout_passes=False, use_tc_tiling_on_sc=False),
      scratch_types=scratch,
  )
  def kern(table_ref, nbt_ref, *rest):
    if trans is not None:
      trans_ref = rest[0]
      out_ref, acc_v, buf_v, ids_v, stage_v, sem, tmap_v = rest[1:]
    else:
      trans_ref = None
      out_ref, acc_v, buf_v, ids_v, stage_v, sem = rest

    wid = lax.axis_index("s") * _NC + lax.axis_index("c")
    base = wid * per
    if trans is not None:
      nb = n_out // B
      b = (wid * per) // nb
      pltpu.sync_copy(trans_ref.at[b, :], tmap_v)

    for c0 in range(0, per, ch):
      start = base + c0
      if trans is None:
        pltpu.sync_copy(table_ref.at[pl.ds(start, ch)], acc_v)
      else:
        p0 = (wid * per) % nb + c0
        pltpu.sync_copy(trans_ref.at[b, pl.ds(p0, ch)], stage_v)
        _vaddupdate_offset(stage_v, ch, b * self_off)
        pltpu.async_copy(table_ref.at[stage_v], acc_v, sem).wait()
      for j in range(k):
        if trans is None:
          pltpu.sync_copy(nbt_ref.at[j, pl.ds(start, ch)], ids_v.at[j])
        else:
          pltpu.sync_copy(nbt_ref.at[j, pl.ds(start, ch)], stage_v)

          def tr_body(cc, carry, j=j):
            sl = pl.ds(cc * 16, 16)
            q = stage_v[sl]
            ids_v[j, sl] = plsc.load_gather(tmap_v, [q]) + b * self_off
            return carry

          lax.fori_loop(0, ch // 16, tr_body, 0)
      copies = [pltpu.async_copy(table_ref.at[ids_v.at[j]], buf_v.at[j], sem)
                for j in range(k)]
      for cp in copies:
        cp.wait()
      for j in range(k):
        _vadd(acc_v, buf_v.at[j], ch, H)
      pltpu.sync_copy(acc_v, out_ref.at[pl.ds(start, ch)])

  if trans is not None:
    return kern(table, nbt, trans)
  return kern(table, nbt)


def kernel(x, edge_score, node_score, fc1_W, fc1_b, fc2_W, fc2_b,
           gnn_W0, gnn_b0, gnn_W1, gnn_b1, gnn_W2, gnn_b2):
  n1, n2 = N // DSN, N // (DSN * DSN)
  k0, k1, k2 = K0, K0 // DSE, K0 // (DSE * DSE)

  key = jax.random.key(42)

  def unif(k_, shape):
    return jax.random.uniform(k_, shape, minval=1e-20, maxval=1.0)

  def gumbel(k_, shape):
    return -jnp.log(-jnp.log(unif(k_, shape)))

  u_e0 = unif(jax.random.fold_in(key, 0), (B * N, N))
  u_e1 = unif(jax.random.fold_in(key, 2), (B * n1, n1))
  u_e2 = unif(jax.random.fold_in(key, 4), (B * n2, n2))
  g_n0 = gumbel(jax.random.fold_in(key, 1), (B, N))
  g_n1 = gumbel(jax.random.fold_in(key, 3), (B, n1))

  ns = jax.nn.softmax(-node_score, axis=-1)
  ln = jnp.where(ns > 0, jnp.log(jnp.maximum(ns, 1e-30)), -jnp.inf)

  i0, i1, sel = _node_select(
      ln.reshape(B, 8, 128),
      g_n0.reshape(B, 8, 128),
      g_n1.reshape(B, 4, 128),
  )

  q0, ls2 = _edge0_topk(edge_score.reshape(B * N, N), u_e0)
  lsg1, lsg2 = _sc_gather_scores(ls2, i0, sel, n1, n2)
  q1 = _edge_topk(lsg1, u_e1, k1)
  q2 = _edge_topk(lsg2, u_e2, k2)

  q0t = q0.T.reshape(k0, B * N)
  q1t = q1.T.reshape(k1, B * n1)
  q2t = q2.T.reshape(k2, B * n2)

  xw0 = _sort_fc1(x.reshape(B * N, S), fc1_W, fc1_b.reshape(1, H), gnn_W0)
  sum0 = _sc_gather_sum(xw0, q0t, k0, B * N)
  xw1 = _scale_mm(sum0, gnn_b0.reshape(1, H), gnn_W1, 1.0 / 9.0)
  sum1 = _sc_gather_sum(xw1, q1t, k1, B * n1,
                        trans=i0, trans_n=n1, self_off=N)
  xw2 = _scale_mm(sum1, gnn_b1.reshape(1, H), gnn_W2, 1.0 / 5.0)
  sum2 = _sc_gather_sum(xw2, q2t, k2, B * n2,
                        trans=i1, trans_n=n2, self_off=n1)

  out = _final(sum2.reshape(B, n2, H), gnn_b2.reshape(1, H),
               fc2_W, fc2_b.reshape(1, 1))
  return out, sel
